# baseline (speedup 1.0000x reference)
import functools

import jax
import jax.numpy as jnp
from jax import lax
from jax.experimental import pallas as pl
from jax.experimental.pallas import tpu as pltpu
from jax.experimental.pallas import tpu_sc as plsc

NW = 32
CHUNK = 128
LANES = 16


def _emb_body(n_chunks, seq, x_hbm, word_hbm, pos_hbm, out_hbm,
              idx_v, word_v, pos_v, sem):
    c = lax.axis_index("c")
    s = lax.axis_index("s")
    wid = s * 2 + c
    rows_per_w = n_chunks * CHUNK
    base = wid * rows_per_w
    pos_base = lax.rem(base, seq)

    pltpu.sync_copy(x_hbm.at[wid], idx_v)

    def chunk_body(j, carry):
        pltpu.async_copy(word_hbm.at[idx_v.at[j]], word_v, sem).wait()
        pltpu.sync_copy(pos_hbm.at[pl.ds(pos_base + j * CHUNK, CHUNK)], pos_v)

        def row_body(i, carry2):
            for h in range(8):
                sl = pl.ds(h * LANES, LANES)
                word_v[i, sl] = word_v[i, sl] + pos_v[i, sl]
            return carry2

        lax.fori_loop(0, CHUNK, row_body, 0)
        pltpu.sync_copy(word_v, out_hbm.at[pl.ds(base + j * CHUNK, CHUNK)])
        return carry

    lax.fori_loop(0, n_chunks, chunk_body, 0)


@functools.partial(jax.jit, static_argnames=())
def _run(x, word_table, pos_table):
    batch, seq = x.shape
    vocab, hidden = word_table.shape
    total = batch * seq
    rows_per_w = total // NW
    n_chunks = rows_per_w // CHUNK

    xr = x.reshape(NW, n_chunks, CHUNK).astype(jnp.int32)

    out = pl.kernel(
        functools.partial(_emb_body, n_chunks, seq),
        out_type=jax.ShapeDtypeStruct((total, hidden), jnp.float32),
        mesh=plsc.VectorSubcoreMesh(core_axis_name="c", subcore_axis_name="s"),
        scratch_types=[
            pltpu.VMEM((n_chunks, CHUNK), jnp.int32),
            pltpu.VMEM((CHUNK, hidden), jnp.float32),
            pltpu.VMEM((CHUNK, hidden), jnp.float32),
            pltpu.SemaphoreType.DMA,
        ],
    )(xr, word_table, pos_table)
    return out.reshape(batch, seq, hidden)


def kernel(x, word_table, pos_table):
    batch, seq = x.shape
    assert (batch * seq) % (NW * CHUNK) == 0
    assert seq % ((batch * seq) // NW) == 0
    return _run(x, word_table, pos_table)


# speedup vs baseline: 1.3350x; 1.3350x over previous
import functools

import jax
import jax.numpy as jnp
from jax import lax
from jax.experimental import pallas as pl
from jax.experimental.pallas import tpu as pltpu
from jax.experimental.pallas import tpu_sc as plsc

NW = 32
CHUNK = 128
NBUF = 2
LANES = 16


def _emb_body(n_chunks, seq, x_hbm, word_hbm, pos_hbm, out_hbm,
              idx_v, word_v, pos_v, gsems, osems, psems):
    c = lax.axis_index("c")
    s = lax.axis_index("s")
    wid = s * 2 + c
    rows_per_w = n_chunks * CHUNK
    base = wid * rows_per_w
    pos_base = lax.rem(base, seq)

    pltpu.sync_copy(x_hbm.at[wid], idx_v)

    def start(j, b):
        g = pltpu.async_copy(word_hbm.at[idx_v.at[j]], word_v.at[b], gsems.at[b])
        p = pltpu.async_copy(
            pos_hbm.at[pl.ds(pos_base + j * CHUNK, CHUNK)], pos_v.at[b],
            psems.at[b])
        return g, p

    pending_out = [None] * NBUF
    pending_in = [None] * NBUF
    pending_in[0] = start(0, 0)

    for j in range(n_chunks):
        b = j % NBUF
        nb = (j + 1) % NBUF
        if j + 1 < n_chunks:
            if pending_out[nb] is not None:
                pending_out[nb].wait()
                pending_out[nb] = None
            pending_in[nb] = start(j + 1, nb)
        g, p = pending_in[b]
        g.wait()
        p.wait()

        def row_body(i, carry):
            for h in range(8):
                sl = pl.ds(h * LANES, LANES)
                plsc.addupdate(word_v.at[b, i, sl], pos_v[b, i, sl])
            return carry

        lax.fori_loop(0, CHUNK, row_body, 0)
        pending_out[b] = pltpu.async_copy(
            word_v.at[b], out_hbm.at[pl.ds(base + j * CHUNK, CHUNK)],
            osems.at[b])

    for b in range(NBUF):
        if pending_out[b] is not None:
            pending_out[b].wait()


@jax.jit
def _run(x, word_table, pos_table):
    batch, seq = x.shape
    vocab, hidden = word_table.shape
    total = batch * seq
    rows_per_w = total // NW
    n_chunks = rows_per_w // CHUNK

    xr = x.reshape(NW, n_chunks, CHUNK).astype(jnp.int32)

    out = pl.kernel(
        functools.partial(_emb_body, n_chunks, seq),
        out_type=jax.ShapeDtypeStruct((total, hidden), jnp.float32),
        mesh=plsc.VectorSubcoreMesh(core_axis_name="c", subcore_axis_name="s"),
        scratch_types=[
            pltpu.VMEM((n_chunks, CHUNK), jnp.int32),
            pltpu.VMEM((NBUF, CHUNK, hidden), jnp.float32),
            pltpu.VMEM((NBUF, CHUNK, hidden), jnp.float32),
            pltpu.SemaphoreType.DMA((NBUF,)),
            pltpu.SemaphoreType.DMA((NBUF,)),
            pltpu.SemaphoreType.DMA((NBUF,)),
        ],
    )(xr, word_table, pos_table)
    return out.reshape(batch, seq, hidden)


def kernel(x, word_table, pos_table):
    batch, seq = x.shape
    assert (batch * seq) % (NW * CHUNK) == 0
    assert seq % ((batch * seq) // NW) == 0
    return _run(x, word_table, pos_table)


# speedup vs baseline: 1.4332x; 1.0735x over previous
import functools

import jax
import jax.numpy as jnp
from jax import lax
from jax.experimental import pallas as pl
from jax.experimental.pallas import tpu as pltpu
from jax.experimental.pallas import tpu_sc as plsc

NW = 32
NS = 16
CHUNK = 128
NBUF = 2
LANES = 16


def _emb_body(n_chunks, seq, x_hbm, word_hbm, pos_hbm, out_hbm,
              idx_v, word_v, pos_v, pos_sh, gsems, osems, psems):
    c = lax.axis_index("c")
    s = lax.axis_index("s")
    rows_per_w = n_chunks * CHUNK
    half = seq // 2
    base = (s // 4) * seq + c * half + lax.rem(s, 4) * rows_per_w
    pos_local = lax.rem(s, 4) * rows_per_w

    pltpu.sync_copy(x_hbm.at[pl.ds(base, rows_per_w)], idx_v)

    def gather(j, b):
        return pltpu.async_copy(
            word_hbm.at[idx_v.at[pl.ds(j * CHUNK, CHUNK)]], word_v.at[b],
            gsems.at[b])

    pending_in = [None] * NBUF
    pending_out = [None] * NBUF
    pending_in[0] = gather(0, 0)

    stage = half // NS
    pltpu.sync_copy(pos_hbm.at[pl.ds(c * half + s * stage, stage)],
                    pos_sh.at[pl.ds(s * stage, stage)])
    plsc.subcore_barrier()

    pending_pos = [None] * NBUF
    pending_pos[0] = pltpu.async_copy(
        pos_sh.at[pl.ds(pos_local, CHUNK)], pos_v.at[0], psems.at[0])

    for j in range(n_chunks):
        b = j % NBUF
        nb = (j + 1) % NBUF
        if j + 1 < n_chunks:
            if pending_out[nb] is not None:
                pending_out[nb].wait()
                pending_out[nb] = None
            pending_in[nb] = gather(j + 1, nb)
            pending_pos[nb] = pltpu.async_copy(
                pos_sh.at[pl.ds(pos_local + (j + 1) * CHUNK, CHUNK)],
                pos_v.at[nb], psems.at[nb])
        pending_in[b].wait()
        pending_pos[b].wait()

        def row_body(i, carry):
            for h in range(8):
                sl = pl.ds(h * LANES, LANES)
                plsc.addupdate(word_v.at[b, i, sl], pos_v[b, i, sl])
            return carry

        lax.fori_loop(0, CHUNK, row_body, 0)
        pending_out[b] = pltpu.async_copy(
            word_v.at[b], out_hbm.at[pl.ds(base + j * CHUNK, CHUNK)],
            osems.at[b])

    for b in range(NBUF):
        if pending_out[b] is not None:
            pending_out[b].wait()


@jax.jit
def _run(x, word_table, pos_table):
    batch, seq = x.shape
    vocab, hidden = word_table.shape
    total = batch * seq
    rows_per_w = total // NW
    n_chunks = rows_per_w // CHUNK

    xf = x.reshape(total).astype(jnp.int32)

    out = pl.kernel(
        functools.partial(_emb_body, n_chunks, seq),
        out_type=jax.ShapeDtypeStruct((total, hidden), jnp.float32),
        mesh=plsc.VectorSubcoreMesh(core_axis_name="c", subcore_axis_name="s"),
        scratch_types=[
            pltpu.VMEM((rows_per_w,), jnp.int32),
            pltpu.VMEM((NBUF, CHUNK, hidden), jnp.float32),
            pltpu.VMEM((NBUF, CHUNK, hidden), jnp.float32),
            pltpu.VMEM_SHARED((seq // 2, hidden), jnp.float32),
            pltpu.SemaphoreType.DMA((NBUF,)),
            pltpu.SemaphoreType.DMA((NBUF,)),
            pltpu.SemaphoreType.DMA((NBUF,)),
        ],
    )(xf, word_table, pos_table)
    return out.reshape(batch, seq, hidden)


def kernel(x, word_table, pos_table):
    batch, seq = x.shape
    assert (batch * seq) % (NW * CHUNK) == 0
    assert batch == NS // 4
    assert (batch * seq) // NW == seq // 8
    assert (seq // 2) % NS == 0
    return _run(x, word_table, pos_table)


# speedup vs baseline: 1.4741x; 1.0285x over previous
import functools

import jax
import jax.numpy as jnp
from jax import lax
from jax.experimental import pallas as pl
from jax.experimental.pallas import tpu as pltpu
from jax.experimental.pallas import tpu_sc as plsc

NW = 32
NS = 16
CHUNK = 128
NBUF = 3
LANES = 16


def _emb_body(n_chunks, seq, x_hbm, word_hbm, pos_hbm, out_hbm,
              idx_v, word_v, pos_v, pos_sh, gsems, osems, psems):
    c = lax.axis_index("c")
    s = lax.axis_index("s")
    rows_per_w = n_chunks * CHUNK
    half = seq // 2
    col0 = c * half + lax.rem(s, 4) * rows_per_w
    base = (s // 4) * seq + col0
    pos_local = lax.rem(s, 4) * rows_per_w

    pltpu.sync_copy(x_hbm.at[s // 4, pl.ds(col0, rows_per_w)], idx_v)

    def gather(j, b):
        return pltpu.async_copy(
            word_hbm.at[idx_v.at[pl.ds(j * CHUNK, CHUNK)]], word_v.at[b],
            gsems.at[b])

    pending_in = [None] * NBUF
    pending_out = [None] * NBUF
    pending_pos = [None] * NBUF
    pending_in[0] = gather(0, 0)
    pending_in[1] = gather(1, 1)

    stage = half // NS
    pltpu.sync_copy(pos_hbm.at[pl.ds(c * half + s * stage, stage)],
                    pos_sh.at[pl.ds(s * stage, stage)])
    plsc.subcore_barrier()

    def pos_fetch(j, b):
        return pltpu.async_copy(
            pos_sh.at[pl.ds(pos_local + j * CHUNK, CHUNK)], pos_v.at[b],
            psems.at[b])

    pending_pos[0] = pos_fetch(0, 0)

    for j in range(n_chunks):
        b = j % NBUF
        pb = j % 2
        nb = (j + 2) % NBUF
        if j + 2 < n_chunks:
            if pending_out[nb] is not None:
                pending_out[nb].wait()
                pending_out[nb] = None
            pending_in[nb] = gather(j + 2, nb)
        if j + 1 < n_chunks:
            pending_pos[(j + 1) % 2] = pos_fetch(j + 1, (j + 1) % 2)
        pending_in[b].wait()
        pending_pos[pb].wait()

        def row_body(i, carry):
            for h in range(8):
                sl = pl.ds(h * LANES, LANES)
                plsc.addupdate(word_v.at[b, i, sl], pos_v[pb, i, sl])
            return carry

        lax.fori_loop(0, CHUNK, row_body, 0)
        pending_out[b] = pltpu.async_copy(
            word_v.at[b], out_hbm.at[pl.ds(base + j * CHUNK, CHUNK)],
            osems.at[b])

    for b in range(NBUF):
        if pending_out[b] is not None:
            pending_out[b].wait()


@jax.jit
def _run(x, word_table, pos_table):
    batch, seq = x.shape
    vocab, hidden = word_table.shape
    total = batch * seq
    rows_per_w = total // NW
    n_chunks = rows_per_w // CHUNK

    xi = x.astype(jnp.int32)

    out = pl.kernel(
        functools.partial(_emb_body, n_chunks, seq),
        out_type=jax.ShapeDtypeStruct((total, hidden), jnp.float32),
        mesh=plsc.VectorSubcoreMesh(core_axis_name="c", subcore_axis_name="s"),
        scratch_types=[
            pltpu.VMEM((rows_per_w,), jnp.int32),
            pltpu.VMEM((NBUF, CHUNK, hidden), jnp.float32),
            pltpu.VMEM((2, CHUNK, hidden), jnp.float32),
            pltpu.VMEM_SHARED((seq // 2, hidden), jnp.float32),
            pltpu.SemaphoreType.DMA((NBUF,)),
            pltpu.SemaphoreType.DMA((NBUF,)),
            pltpu.SemaphoreType.DMA((2,)),
        ],
    )(xi, word_table, pos_table)
    return out.reshape(batch, seq, hidden)


def kernel(x, word_table, pos_table):
    batch, seq = x.shape
    assert (batch * seq) % (NW * CHUNK) == 0
    assert batch == NS // 4
    assert (batch * seq) // NW == seq // 8
    assert (seq // 2) % NS == 0
    return _run(x, word_table, pos_table)
